# baseline (speedup 1.0000x reference)
import functools

import jax
import jax.numpy as jnp
from jax import lax
from jax.experimental import pallas as pl
from jax.experimental.pallas import tpu as pltpu
from jax.experimental.pallas import tpu_sc as plsc

NUM_CORES = 2
NUM_SUBCORES = 16
NUM_WORKERS = NUM_CORES * NUM_SUBCORES
LANES = 16
NBUF = 6
UNROLL = 2
AHEAD = 4

CHUNK_OFF = (0, 96)
CHUNK_LEN = (96, 104)
MAXC = 104


def _body(B, L, D, x_hbm, tab_hbm, pos_hbm, out_hbm, idx_all, buf, pos_v,
          gsems, osems, isem, psem):
    cid = lax.axis_index("c")
    sid = lax.axis_index("s")
    wid = sid * NUM_CORES + cid
    seq_per_w = B // NUM_WORKERS
    dreg = D // LANES
    b0 = wid * seq_per_w
    n_units = 2 * seq_per_w

    pltpu.async_copy(x_hbm.at[pl.ds(b0 * L, seq_per_w * L)], idx_all, isem)
    pltpu.async_copy(pos_hbm, pos_v, psem)

    def start_gather(s, parity, slot):
        off = CHUNK_OFF[parity]
        n = CHUNK_LEN[parity]
        pltpu.async_copy(
            tab_hbm.at[idx_all.at[pl.ds(s * L + off, n)]],
            buf.at[slot, pl.ds(0, n)],
            gsems[slot],
        )

    def wait_gather(parity, slot):
        n = CHUNK_LEN[parity]
        pltpu.make_async_copy(
            tab_hbm.at[idx_all.at[pl.ds(0, n)]],
            buf.at[slot, pl.ds(0, n)],
            gsems[slot],
        ).wait()

    def wait_out(parity, slot):
        off = CHUNK_OFF[parity]
        n = CHUNK_LEN[parity]
        pltpu.make_async_copy(
            buf.at[slot, pl.ds(0, n)],
            out_hbm.at[b0, pl.ds(off, n)],
            osems[slot],
        ).wait()

    def step(s, r):
        parity = r % 2
        off = CHUNK_OFF[parity]
        n = CHUNK_LEN[parity]
        nslot = (r + AHEAD) % NBUF

        u = 2 * s + parity

        @pl.when(u + AHEAD < n_units)
        def _():
            @pl.when(u >= NBUF - AHEAD)
            def _():
                wait_out(parity, nslot)

            start_gather(s + AHEAD // 2, parity, nslot)

        wait_gather(parity, r)

        @pl.loop(0, n, step=UNROLL)
        def _row(i):
            for q in range(UNROLL):
                for d in range(dreg):
                    sl = pl.ds(d * LANES, LANES)
                    buf[r, i + q, sl] = buf[r, i + q, sl] + pos_v[off + i + q, sl]

        pltpu.async_copy(
            buf.at[r, pl.ds(0, n)],
            out_hbm.at[b0 + s, pl.ds(off, n)],
            osems[r],
        )

    pltpu.make_async_copy(
        x_hbm.at[pl.ds(b0 * L, seq_per_w * L)], idx_all, isem
    ).wait()
    for v in range(AHEAD):
        start_gather(v // 2, v % 2, v)
    pltpu.make_async_copy(pos_hbm, pos_v, psem).wait()

    main_units = (n_units // NBUF) * NBUF

    @pl.loop(0, main_units // 2, step=NBUF // 2)
    def _s(s0):
        for r in range(NBUF):
            step(s0 + r // 2, r)

    for j in range(main_units, n_units):
        step(j // 2, j % NBUF)

    for j in range(n_units - NBUF, n_units):
        wait_out(j % 2, j % NBUF)


def kernel(x, token_table, pos_table):
    B, L = x.shape
    V, D = token_table.shape
    x = x.astype(jnp.int32).reshape(B * L)
    mesh = plsc.VectorSubcoreMesh(
        core_axis_name="c", subcore_axis_name="s", num_cores=NUM_CORES,
        num_subcores=NUM_SUBCORES,
    )
    seq_per_w = B // NUM_WORKERS
    body = functools.partial(_body, B, L, D)
    f = pl.kernel(
        body,
        out_type=jax.ShapeDtypeStruct((B, L, D), jnp.float32),
        mesh=mesh,
        scratch_types=[
            pltpu.VMEM((seq_per_w * L,), jnp.int32),
            pltpu.VMEM((NBUF, MAXC, D), jnp.float32),
            pltpu.VMEM((L, D), jnp.float32),
            [pltpu.SemaphoreType.DMA] * NBUF,
            [pltpu.SemaphoreType.DMA] * NBUF,
            pltpu.SemaphoreType.DMA,
            pltpu.SemaphoreType.DMA,
        ],
    )
    return f(x, token_table, pos_table)


# speedup vs baseline: 1.0136x; 1.0136x over previous
import functools

import jax
import jax.numpy as jnp
from jax import lax
from jax.experimental import pallas as pl
from jax.experimental.pallas import tpu as pltpu
from jax.experimental.pallas import tpu_sc as plsc

NUM_CORES = 2
NUM_SUBCORES = 16
NUM_WORKERS = NUM_CORES * NUM_SUBCORES
LANES = 16
NBUF = 4
UNROLL = 1

CHUNK_OFF = (0, 96)
CHUNK_LEN = (96, 104)
MAXC = 104


def _body(B, L, D, x_hbm, tab_hbm, pos_hbm, out_hbm, idx_all, buf, pos_v,
          gsems, osems, isem, psem):
    cid = lax.axis_index("c")
    sid = lax.axis_index("s")
    wid = sid * NUM_CORES + cid
    seq_per_w = B // NUM_WORKERS
    dreg = D // LANES
    b0 = wid * seq_per_w
    n_units = 2 * seq_per_w

    pltpu.async_copy(x_hbm.at[pl.ds(b0 * L, seq_per_w * L)], idx_all, isem)
    pltpu.async_copy(pos_hbm, pos_v, psem)

    def start_gather(s, parity, slot):
        off = CHUNK_OFF[parity]
        n = CHUNK_LEN[parity]
        pltpu.async_copy(
            tab_hbm.at[idx_all.at[pl.ds(s * L + off, n)]],
            buf.at[slot, pl.ds(0, n)],
            gsems[slot],
        )

    def wait_gather(parity, slot):
        n = CHUNK_LEN[parity]
        pltpu.make_async_copy(
            tab_hbm.at[idx_all.at[pl.ds(0, n)]],
            buf.at[slot, pl.ds(0, n)],
            gsems[slot],
        ).wait()

    def wait_out(parity, slot):
        off = CHUNK_OFF[parity]
        n = CHUNK_LEN[parity]
        pltpu.make_async_copy(
            buf.at[slot, pl.ds(0, n)],
            out_hbm.at[b0, pl.ds(off, n)],
            osems[slot],
        ).wait()

    def step(s, r):
        parity = r % 2
        off = CHUNK_OFF[parity]
        n = CHUNK_LEN[parity]
        nslot = (r + 2) % NBUF

        u = 2 * s + parity

        @pl.when(u + 2 < n_units)
        def _():
            @pl.when(u >= 2)
            def _():
                wait_out(parity, nslot)

            start_gather(s + 1, parity, nslot)

        wait_gather(parity, r)

        @pl.loop(0, n, step=UNROLL)
        def _row(i):
            for q in range(UNROLL):
                for d in range(dreg):
                    sl = pl.ds(d * LANES, LANES)
                    buf[r, i + q, sl] = buf[r, i + q, sl] + pos_v[off + i + q, sl]

        pltpu.async_copy(
            buf.at[r, pl.ds(0, n)],
            out_hbm.at[b0 + s, pl.ds(off, n)],
            osems[r],
        )

    pltpu.make_async_copy(
        x_hbm.at[pl.ds(b0 * L, seq_per_w * L)], idx_all, isem
    ).wait()
    start_gather(0, 0, 0)
    start_gather(0, 1, 1)
    pltpu.make_async_copy(pos_hbm, pos_v, psem).wait()

    @pl.loop(0, seq_per_w, step=2)
    def _s(s0):
        for r in range(NBUF):
            step(s0 + r // 2, r)

    for r in range(NBUF):
        wait_out(r % 2, r)


def kernel(x, token_table, pos_table):
    B, L = x.shape
    V, D = token_table.shape
    x = x.astype(jnp.int32).reshape(B * L)
    mesh = plsc.VectorSubcoreMesh(
        core_axis_name="c", subcore_axis_name="s", num_cores=NUM_CORES,
        num_subcores=NUM_SUBCORES,
    )
    seq_per_w = B // NUM_WORKERS
    body = functools.partial(_body, B, L, D)
    f = pl.kernel(
        body,
        out_type=jax.ShapeDtypeStruct((B, L, D), jnp.float32),
        mesh=mesh,
        scratch_types=[
            pltpu.VMEM((seq_per_w * L,), jnp.int32),
            pltpu.VMEM((NBUF, MAXC, D), jnp.float32),
            pltpu.VMEM((L, D), jnp.float32),
            [pltpu.SemaphoreType.DMA] * NBUF,
            [pltpu.SemaphoreType.DMA] * NBUF,
            pltpu.SemaphoreType.DMA,
            pltpu.SemaphoreType.DMA,
        ],
    )
    return f(x, token_table, pos_table)


# speedup vs baseline: 1.0171x; 1.0035x over previous
import functools

import jax
import jax.numpy as jnp
from jax import lax
from jax.experimental import pallas as pl
from jax.experimental.pallas import tpu as pltpu
from jax.experimental.pallas import tpu_sc as plsc

NUM_CORES = 2
NUM_SUBCORES = 16
NUM_WORKERS = NUM_CORES * NUM_SUBCORES
LANES = 16
NBUF = 4
UNROLL = 2

CHUNK_OFF = (0, 96)
CHUNK_LEN = (96, 104)
MAXC = 104


def _body(B, L, D, x_hbm, tab_hbm, pos_hbm, out_hbm, idx_all, buf, pos_v,
          gsems, osems, isem, psem):
    cid = lax.axis_index("c")
    sid = lax.axis_index("s")
    wid = sid * NUM_CORES + cid
    seq_per_w = B // NUM_WORKERS
    dreg = D // LANES
    b0 = wid * seq_per_w
    n_units = 2 * seq_per_w

    pltpu.async_copy(x_hbm.at[pl.ds(b0 * L, seq_per_w * L)], idx_all, isem)
    pltpu.async_copy(pos_hbm, pos_v, psem)

    def start_gather(s, parity, slot):
        off = CHUNK_OFF[parity]
        n = CHUNK_LEN[parity]
        pltpu.async_copy(
            tab_hbm.at[idx_all.at[pl.ds(s * L + off, n)]],
            buf.at[slot, pl.ds(0, n)],
            gsems[slot],
        )

    def wait_gather(parity, slot):
        n = CHUNK_LEN[parity]
        pltpu.make_async_copy(
            tab_hbm.at[idx_all.at[pl.ds(0, n)]],
            buf.at[slot, pl.ds(0, n)],
            gsems[slot],
        ).wait()

    def wait_out(parity, slot):
        off = CHUNK_OFF[parity]
        n = CHUNK_LEN[parity]
        pltpu.make_async_copy(
            buf.at[slot, pl.ds(0, n)],
            out_hbm.at[b0, pl.ds(off, n)],
            osems[slot],
        ).wait()

    def step(s, r):
        parity = r % 2
        off = CHUNK_OFF[parity]
        n = CHUNK_LEN[parity]
        nslot = (r + 2) % NBUF

        u = 2 * s + parity

        @pl.when(u + 2 < n_units)
        def _():
            @pl.when(u >= 2)
            def _():
                wait_out(parity, nslot)

            start_gather(s + 1, parity, nslot)

        wait_gather(parity, r)

        @pl.loop(0, n, step=UNROLL)
        def _row(i):
            for q in range(UNROLL):
                for d in range(dreg):
                    sl = pl.ds(d * LANES, LANES)
                    buf[r, i + q, sl] = buf[r, i + q, sl] + pos_v[off + i + q, sl]

        pltpu.async_copy(
            buf.at[r, pl.ds(0, n)],
            out_hbm.at[b0 + s, pl.ds(off, n)],
            osems[r],
        )

    pltpu.make_async_copy(
        x_hbm.at[pl.ds(b0 * L, seq_per_w * L)], idx_all, isem
    ).wait()
    start_gather(0, 0, 0)
    start_gather(0, 1, 1)
    pltpu.make_async_copy(pos_hbm, pos_v, psem).wait()

    @pl.loop(0, seq_per_w, step=2)
    def _s(s0):
        for r in range(NBUF):
            step(s0 + r // 2, r)

    for r in range(NBUF):
        wait_out(r % 2, r)


def kernel(x, token_table, pos_table):
    B, L = x.shape
    V, D = token_table.shape
    x = x.astype(jnp.int32).reshape(B * L)
    mesh = plsc.VectorSubcoreMesh(
        core_axis_name="c", subcore_axis_name="s", num_cores=NUM_CORES,
        num_subcores=NUM_SUBCORES,
    )
    seq_per_w = B // NUM_WORKERS
    body = functools.partial(_body, B, L, D)
    f = pl.kernel(
        body,
        out_type=jax.ShapeDtypeStruct((B, L, D), jnp.float32),
        mesh=mesh,
        scratch_types=[
            pltpu.VMEM((seq_per_w * L,), jnp.int32),
            pltpu.VMEM((NBUF, MAXC, D), jnp.float32),
            pltpu.VMEM((L, D), jnp.float32),
            [pltpu.SemaphoreType.DMA] * NBUF,
            [pltpu.SemaphoreType.DMA] * NBUF,
            pltpu.SemaphoreType.DMA,
            pltpu.SemaphoreType.DMA,
        ],
    )
    return f(x, token_table, pos_table)
